# Initial kernel scaffold; baseline (speedup 1.0000x reference)
import functools

import jax
import jax.numpy as jnp
from jax import lax
from jax.experimental import pallas as pl
from jax.experimental.pallas import tpu as pltpu
from jax.experimental.pallas import tpu_sc as plsc

_NMS_THR = 0.3
_MAX_OUT = 320


def _nms_body(scores_ref, x1_ref, y1_ref, x2_ref, y2_ref, keep_ref, s_ref, a_ref):
    x1 = x1_ref[...]
    y1 = y1_ref[...]
    x2 = x2_ref[...]
    y2 = y2_ref[...]
    areas = jnp.maximum(x2 - x1, 0.0) * jnp.maximum(y2 - y1, 0.0)
    a_ref[...] = areas
    s_ref[...] = scores_ref[...]
    keep_ref[...] = jnp.zeros_like(keep_ref)
    rows, cols = s_ref.shape
    col = lax.broadcasted_iota(jnp.int32, (rows, cols), 1)

    def body(_):
        s = s_ref[...]
        a = a_ref[...]
        m = jnp.max(s, axis=1, keepdims=True)
        active = m > -0.5
        idxm = jnp.min(jnp.where(s == m, col, cols), axis=1, keepdims=True)
        cand = (col == idxm) & active
        cf = cand.astype(jnp.float32)
        cx1 = jnp.sum(x1 * cf, axis=1, keepdims=True)
        cy1 = jnp.sum(y1 * cf, axis=1, keepdims=True)
        cx2 = jnp.sum(x2 * cf, axis=1, keepdims=True)
        cy2 = jnp.sum(y2 * cf, axis=1, keepdims=True)
        ca = jnp.sum(a * cf, axis=1, keepdims=True)
        xx1 = jnp.maximum(x1, cx1)
        yy1 = jnp.maximum(y1, cy1)
        xx2 = jnp.minimum(x2, cx2)
        yy2 = jnp.minimum(y2, cy2)
        inter = jnp.maximum(xx2 - xx1, 0.0) * jnp.maximum(yy2 - yy1, 0.0)
        union = ca + a - inter
        iou = inter / jnp.maximum(union, 1e-9)
        supp = ((iou > _NMS_THR) & active) | cand
        s_new = jnp.where(supp, -1.0, s)
        s_ref[...] = s_new
        keep_ref[...] = keep_ref[...] + cf
        return jnp.any(s_new > -0.5)

    lax.while_loop(lambda c: c, body, jnp.any(s_ref[...] > -0.5))


def _compact_body(keep_hbm, cls_hbm, x1_hbm, y1_hbm, x2_hbm, y2_hbm, sc_hbm,
                  oc_hbm, o1_hbm, o2_hbm, o3_hbm, o4_hbm, os_hbm,
                  k_s, c_s, x1_s, y1_s, x2_s, y2_s, s_s,
                  oc_s, o1_s, o2_s, o3_s, o4_s, os_s):
    nb = keep_hbm.shape[0]
    npad = keep_hbm.shape[1]
    wid = lax.axis_index("s") * 2 + lax.axis_index("c")

    @pl.when(wid < nb)
    def _():
        pltpu.sync_copy(keep_hbm.at[wid], k_s)
        pltpu.sync_copy(cls_hbm.at[wid], c_s)
        pltpu.sync_copy(x1_hbm.at[wid], x1_s)
        pltpu.sync_copy(y1_hbm.at[wid], y1_s)
        pltpu.sync_copy(x2_hbm.at[wid], x2_s)
        pltpu.sync_copy(y2_hbm.at[wid], y2_s)
        pltpu.sync_copy(sc_hbm.at[wid], s_s)

        outs = (oc_s, o1_s, o2_s, o3_s, o4_s, os_s)
        srcs = (c_s, x1_s, y1_s, x2_s, y2_s, s_s)

        def zero(i, _):
            z = jnp.zeros((16,), jnp.float32)
            for oref in outs:
                oref[pl.ds(i * 16, 16)] = z
            return 0

        lax.fori_loop(0, _MAX_OUT // 16, zero, 0)

        def step(i, base):
            kv = k_s[pl.ds(i * 16, 16)]
            ci = plsc.cumsum(kv)
            pos = base + ci.astype(jnp.int32) - 1
            msk = (kv > 0.5) & (pos < _MAX_OUT)
            for src, dst in zip(srcs, outs):
                plsc.store_scatter(dst, [pos], src[pl.ds(i * 16, 16)], msk)
            return base + jnp.sum(kv).astype(jnp.int32)

        lax.fori_loop(0, npad // 16, step, jnp.int32(0))

        pltpu.sync_copy(oc_s, oc_hbm.at[wid])
        pltpu.sync_copy(o1_s, o1_hbm.at[wid])
        pltpu.sync_copy(o2_s, o2_hbm.at[wid])
        pltpu.sync_copy(o3_s, o3_hbm.at[wid])
        pltpu.sync_copy(o4_s, o4_hbm.at[wid])
        pltpu.sync_copy(os_s, os_hbm.at[wid])


@functools.partial(jax.jit, static_argnums=(5,))
def _run_nms(scores8, x18, y18, x28, y28, npad):
    return pl.pallas_call(
        _nms_body,
        out_shape=jax.ShapeDtypeStruct((8, npad), jnp.float32),
        scratch_shapes=[
            pltpu.VMEM((8, npad), jnp.float32),
            pltpu.VMEM((8, npad), jnp.float32),
        ],
    )(scores8, x18, y18, x28, y28)


@functools.partial(jax.jit, static_argnums=(7, 8))
def _run_compact(keep, cls_a, x1, y1, x2, y2, sc_a, nb, npad):
    mesh = plsc.VectorSubcoreMesh(core_axis_name="c", subcore_axis_name="s")
    out_type = [jax.ShapeDtypeStruct((nb, _MAX_OUT), jnp.float32)] * 6
    scratch = [pltpu.VMEM((npad,), jnp.float32)] * 7 + \
              [pltpu.VMEM((_MAX_OUT,), jnp.float32)] * 6
    return pl.kernel(
        _compact_body,
        out_type=out_type,
        mesh=mesh,
        scratch_types=scratch,
    )(keep, cls_a, x1, y1, x2, y2, sc_a)


def kernel(cls_proposals, gt_classes, box_proposals, gt_boxes, proposal_scores):
    nb = gt_boxes.shape[0]
    cls_all = jnp.concatenate([gt_classes, cls_proposals], axis=1)
    box_all = jnp.concatenate([gt_boxes, box_proposals], axis=1)
    sc_all = jnp.concatenate([gt_classes, proposal_scores], axis=1)
    n = box_all.shape[1]
    npad = ((n + 511) // 512) * 512

    x1 = box_all[:, :, 0]
    y1 = box_all[:, :, 1]
    x2 = box_all[:, :, 2]
    y2 = box_all[:, :, 3]

    def pad_rows(arr, value):
        out = jnp.full((8, npad), value, jnp.float32)
        return out.at[:nb, :n].set(arr)

    scores8 = pad_rows(sc_all, -1.0)
    x18 = pad_rows(x1, 0.0)
    y18 = pad_rows(y1, 0.0)
    x28 = pad_rows(x2, 0.0)
    y28 = pad_rows(y2, 0.0)

    keep = _run_nms(scores8, x18, y18, x28, y28, npad)[:nb]

    def pad_cols(arr):
        return jnp.pad(arr, ((0, 0), (0, npad - n)))

    oc, o1, o2, o3, o4, osc = _run_compact(
        keep, pad_cols(cls_all), pad_cols(x1), pad_cols(y1), pad_cols(x2),
        pad_cols(y2), pad_cols(sc_all), nb, npad)

    outb = jnp.stack([o1, o2, o3, o4], axis=-1)
    return oc, outb, osc


# speedup vs baseline: 36.2479x; 36.2479x over previous
import functools

import jax
import jax.numpy as jnp
from jax import lax
from jax.experimental import pallas as pl
from jax.experimental.pallas import tpu as pltpu
from jax.experimental.pallas import tpu_sc as plsc

_NMS_THR = 0.3
_MAX_OUT = 320


def _nms_body(scores_ref, x1_ref, y1_ref, x2_ref, y2_ref, keep_ref, s_ref, a_ref):
    x1 = x1_ref[...]
    y1 = y1_ref[...]
    x2 = x2_ref[...]
    y2 = y2_ref[...]
    areas = jnp.maximum(x2 - x1, 0.0) * jnp.maximum(y2 - y1, 0.0)
    a_ref[...] = areas
    s_ref[...] = scores_ref[...]
    keep_ref[...] = jnp.zeros_like(keep_ref)
    rows, cols = s_ref.shape
    col = lax.broadcasted_iota(jnp.int32, (rows, cols), 1)

    def body(_):
        s = s_ref[...]
        a = a_ref[...]
        m = jnp.max(s, axis=1, keepdims=True)
        active = m > -0.5
        idxm = jnp.min(jnp.where(s == m, col, cols), axis=1, keepdims=True)
        cand = (col == idxm) & active
        cf = cand.astype(jnp.float32)
        cx1 = jnp.sum(x1 * cf, axis=1, keepdims=True)
        cy1 = jnp.sum(y1 * cf, axis=1, keepdims=True)
        cx2 = jnp.sum(x2 * cf, axis=1, keepdims=True)
        cy2 = jnp.sum(y2 * cf, axis=1, keepdims=True)
        ca = jnp.sum(a * cf, axis=1, keepdims=True)
        xx1 = jnp.maximum(x1, cx1)
        yy1 = jnp.maximum(y1, cy1)
        xx2 = jnp.minimum(x2, cx2)
        yy2 = jnp.minimum(y2, cy2)
        inter = jnp.maximum(xx2 - xx1, 0.0) * jnp.maximum(yy2 - yy1, 0.0)
        union = ca + a - inter
        iou = inter / jnp.maximum(union, 1e-9)
        supp = ((iou > _NMS_THR) & active) | cand
        s_new = jnp.where(supp, -1.0, s)
        s_ref[...] = s_new
        keep_ref[...] = keep_ref[...] + cf
        return jnp.any(s_new > -0.5)

    lax.while_loop(lambda c: c, body, jnp.any(s_ref[...] > -0.5))


def _compact_body(keep_hbm, cls_hbm, x1_hbm, y1_hbm, x2_hbm, y2_hbm, sc_hbm,
                  oc_hbm, o1_hbm, o2_hbm, o3_hbm, o4_hbm, os_hbm,
                  k_s, c_s, x1_s, y1_s, x2_s, y2_s, s_s,
                  oc_s, o1_s, o2_s, o3_s, o4_s, os_s):
    nb = keep_hbm.shape[0]
    npad = keep_hbm.shape[1]
    wid = lax.axis_index("s") * 2 + lax.axis_index("c")

    @pl.when(wid < nb)
    def _():
        pltpu.sync_copy(keep_hbm.at[wid], k_s)
        pltpu.sync_copy(cls_hbm.at[wid], c_s)
        pltpu.sync_copy(x1_hbm.at[wid], x1_s)
        pltpu.sync_copy(y1_hbm.at[wid], y1_s)
        pltpu.sync_copy(x2_hbm.at[wid], x2_s)
        pltpu.sync_copy(y2_hbm.at[wid], y2_s)
        pltpu.sync_copy(sc_hbm.at[wid], s_s)

        outs = (oc_s, o1_s, o2_s, o3_s, o4_s, os_s)
        srcs = (c_s, x1_s, y1_s, x2_s, y2_s, s_s)

        def zero(i, _):
            z = jnp.zeros((16,), jnp.float32)
            for oref in outs:
                oref[pl.ds(i * 16, 16)] = z
            return 0

        lax.fori_loop(0, _MAX_OUT // 16, zero, 0)

        def step(i, base):
            kv = k_s[pl.ds(i * 16, 16)]
            ci = plsc.cumsum(kv)
            pos = base + ci.astype(jnp.int32) - 1
            msk = (kv > 0.5) & (pos < _MAX_OUT)
            for src, dst in zip(srcs, outs):
                plsc.store_scatter(dst, [pos], src[pl.ds(i * 16, 16)], mask=msk)
            return base + jnp.sum(kv).astype(jnp.int32)

        lax.fori_loop(0, npad // 16, step, jnp.int32(0))

        pltpu.sync_copy(oc_s, oc_hbm.at[wid])
        pltpu.sync_copy(o1_s, o1_hbm.at[wid])
        pltpu.sync_copy(o2_s, o2_hbm.at[wid])
        pltpu.sync_copy(o3_s, o3_hbm.at[wid])
        pltpu.sync_copy(o4_s, o4_hbm.at[wid])
        pltpu.sync_copy(os_s, os_hbm.at[wid])


@functools.partial(jax.jit, static_argnums=(5,))
def _run_nms(scores8, x18, y18, x28, y28, npad):
    return pl.pallas_call(
        _nms_body,
        out_shape=jax.ShapeDtypeStruct((8, npad), jnp.float32),
        scratch_shapes=[
            pltpu.VMEM((8, npad), jnp.float32),
            pltpu.VMEM((8, npad), jnp.float32),
        ],
    )(scores8, x18, y18, x28, y28)


@functools.partial(jax.jit, static_argnums=(7, 8))
def _run_compact(keep, cls_a, x1, y1, x2, y2, sc_a, nb, npad):
    mesh = plsc.VectorSubcoreMesh(core_axis_name="c", subcore_axis_name="s")
    out_type = [jax.ShapeDtypeStruct((nb, _MAX_OUT), jnp.float32)] * 6
    scratch = [pltpu.VMEM((npad,), jnp.float32)] * 7 + \
              [pltpu.VMEM((_MAX_OUT,), jnp.float32)] * 6
    return pl.kernel(
        _compact_body,
        out_type=out_type,
        mesh=mesh,
        scratch_types=scratch,
        compiler_params=pltpu.CompilerParams(needs_layout_passes=False),
    )(keep, cls_a, x1, y1, x2, y2, sc_a)


def kernel(cls_proposals, gt_classes, box_proposals, gt_boxes, proposal_scores):
    nb = gt_boxes.shape[0]
    cls_all = jnp.concatenate([gt_classes, cls_proposals], axis=1)
    box_all = jnp.concatenate([gt_boxes, box_proposals], axis=1)
    sc_all = jnp.concatenate([gt_classes, proposal_scores], axis=1)
    n = box_all.shape[1]
    npad = ((n + 511) // 512) * 512

    x1 = box_all[:, :, 0]
    y1 = box_all[:, :, 1]
    x2 = box_all[:, :, 2]
    y2 = box_all[:, :, 3]

    def pad_rows(arr, value):
        out = jnp.full((8, npad), value, jnp.float32)
        return out.at[:nb, :n].set(arr)

    scores8 = pad_rows(sc_all, -1.0)
    x18 = pad_rows(x1, 0.0)
    y18 = pad_rows(y1, 0.0)
    x28 = pad_rows(x2, 0.0)
    y28 = pad_rows(y2, 0.0)

    keep = _run_nms(scores8, x18, y18, x28, y28, npad)[:nb]

    def pad_cols(arr):
        return jnp.pad(arr, ((0, 0), (0, npad - n)))

    oc, o1, o2, o3, o4, osc = _run_compact(
        keep, pad_cols(cls_all), pad_cols(x1), pad_cols(y1), pad_cols(x2),
        pad_cols(y2), pad_cols(sc_all), nb, npad)

    outb = jnp.stack([o1, o2, o3, o4], axis=-1)
    return oc, outb, osc
